# Initial kernel scaffold; baseline (speedup 1.0000x reference)
import jax
import jax.numpy as jnp
from jax.experimental import pallas as pl


def kernel(points_coords, centers_coords, centers_features, points_features, W, b, gamma, beta):
    raise NotImplementedError("write your pallas kernel here")


# speedup vs baseline: 26.2243x; 26.2243x over previous
import jax
import jax.numpy as jnp
from jax.experimental import pallas as pl
from jax.experimental.pallas import tpu as pltpu

B, N, M, CC, CP, COUT = 8, 8192, 1024, 256, 256, 256
CIN = CC + CP
BN = 512
NB = N // BN
BN2 = 2048
NB2 = N // BN2


def _fuse_kernel(pc_ref, ccT_ref, cf_ref, pf_ref, w_ref, b_ref,
                 y_ref, ps_ref, pss_ref, wcf_ref):
    nb = pl.program_id(1)

    @pl.when(nb == 0)
    def _():
        wcf_ref[...] = jnp.dot(w_ref[:, :CC], cf_ref[0],
                               preferred_element_type=jnp.float32)

    c3 = ccT_ref[0]
    p = pc_ref[0]
    p2 = jnp.zeros((1, BN), jnp.float32)
    c2 = jnp.zeros((M, 1), jnp.float32)
    cross = jnp.zeros((M, BN), jnp.float32)
    for d in range(3):
        cd = c3[:, d:d + 1]
        pd = p[d:d + 1, :]
        p2 = p2 + pd * pd
        c2 = c2 + cd * cd
        cdb = cd.astype(jnp.bfloat16).astype(jnp.float32)
        pdb = pd.astype(jnp.bfloat16).astype(jnp.float32)
        cross = cross + cdb * pdb
    d2 = jnp.maximum((p2 + c2) - 2.0 * cross, 0.0)

    iota_m = jax.lax.broadcasted_iota(jnp.int32, (M, BN), 0)
    inf = jnp.float32(3.0e38)
    cur = d2
    s_mat = jnp.zeros((M, BN), jnp.float32)
    wsum = jnp.zeros((1, BN), jnp.float32)
    for k in range(3):
        v = jnp.min(cur, axis=0, keepdims=True)
        i = jnp.min(jnp.where(cur == v, iota_m, M), axis=0,
                    keepdims=True)
        hit = iota_m == i
        wk = 1.0 / (jnp.sqrt(jnp.maximum(v, 1e-12)) + 1e-8)
        s_mat = s_mat + jnp.where(hit, wk, 0.0)
        wsum = wsum + wk
        if k < 2:
            cur = jnp.where(hit, inf, cur)

    y = jnp.dot(wcf_ref[...], s_mat, preferred_element_type=jnp.float32)
    y = y / wsum
    y = y + jnp.dot(w_ref[:, CC:], pf_ref[0],
                    preferred_element_type=jnp.float32)
    y = y + b_ref[...]
    y_ref[0] = y
    ps_ref[0] = jnp.sum(y, axis=1, keepdims=True)
    pss_ref[0] = jnp.sum(y * y, axis=1, keepdims=True)


def _bn_kernel(y_ref, a_ref, c_ref, o_ref):
    o_ref[0] = jnp.maximum(y_ref[0] * a_ref[...] + c_ref[...], 0.0)


def kernel(points_coords, centers_coords, centers_features, points_features,
           W, b, gamma, beta):
    ccT = centers_coords.transpose(0, 2, 1)
    b2 = b.reshape(COUT, 1)

    y, ps, pss = pl.pallas_call(
        _fuse_kernel,
        grid=(B, NB),
        in_specs=[
            pl.BlockSpec((1, 3, BN), lambda bi, nb: (bi, 0, nb)),
            pl.BlockSpec((1, M, 3), lambda bi, nb: (bi, 0, 0)),
            pl.BlockSpec((1, CC, M), lambda bi, nb: (bi, 0, 0)),
            pl.BlockSpec((1, CP, BN), lambda bi, nb: (bi, 0, nb)),
            pl.BlockSpec((COUT, CIN), lambda bi, nb: (0, 0)),
            pl.BlockSpec((COUT, 1), lambda bi, nb: (0, 0)),
        ],
        out_specs=[
            pl.BlockSpec((1, COUT, BN), lambda bi, nb: (bi, 0, nb)),
            pl.BlockSpec((1, COUT, 1), lambda bi, nb: (bi * NB + nb, 0, 0)),
            pl.BlockSpec((1, COUT, 1), lambda bi, nb: (bi * NB + nb, 0, 0)),
        ],
        out_shape=[
            jax.ShapeDtypeStruct((B, COUT, N), jnp.float32),
            jax.ShapeDtypeStruct((B * NB, COUT, 1), jnp.float32),
            jax.ShapeDtypeStruct((B * NB, COUT, 1), jnp.float32),
        ],
        scratch_shapes=[pltpu.VMEM((COUT, M), jnp.float32)],
    )(points_coords, ccT, centers_features, points_features, W, b2)

    cnt = jnp.float32(B * N)
    s = jnp.sum(ps[:, :, 0], axis=0)
    ss = jnp.sum(pss[:, :, 0], axis=0)
    mean = s / cnt
    var = ss / cnt - mean * mean
    a = gamma / jnp.sqrt(var + 1e-5)
    cshift = beta - mean * a

    out = pl.pallas_call(
        _bn_kernel,
        grid=(B, NB2),
        in_specs=[
            pl.BlockSpec((1, COUT, BN2), lambda bi, nb: (bi, 0, nb)),
            pl.BlockSpec((COUT, 1), lambda bi, nb: (0, 0)),
            pl.BlockSpec((COUT, 1), lambda bi, nb: (0, 0)),
        ],
        out_specs=pl.BlockSpec((1, COUT, BN2), lambda bi, nb: (bi, 0, nb)),
        out_shape=jax.ShapeDtypeStruct((B, COUT, N), jnp.float32),
    )(y, a.reshape(COUT, 1), cshift.reshape(COUT, 1))

    return (out, points_coords)
